# Initial kernel scaffold; baseline (speedup 1.0000x reference)
import jax
import jax.numpy as jnp
from jax.experimental import pallas as pl


def kernel(idx, emb_weight):
    raise NotImplementedError("write your pallas kernel here")


# speedup vs baseline: 1.7996x; 1.7996x over previous
import functools

import jax
import jax.numpy as jnp
from jax import lax
from jax.experimental import pallas as pl
from jax.experimental.pallas import tpu as pltpu
from jax.experimental.pallas import tpu_sc as plsc

B = 1024
L = 200
D = 128
N = B * L
NC = 2
NS = 16
NW = NC * NS
PER_W = N // NW
CHUNK = 128
NCH = PER_W // CHUNK
NBUF = 5
NGROUPS = NCH // NBUF

_mesh = plsc.VectorSubcoreMesh(core_axis_name="c", subcore_axis_name="s")


@functools.partial(
    pl.kernel,
    out_type=jax.ShapeDtypeStruct((N, D), jnp.float32),
    mesh=_mesh,
    scratch_types=[
        pltpu.VMEM((NCH, CHUNK), jnp.int32),
        pltpu.VMEM((NBUF, CHUNK, D), jnp.float32),
    ]
    + [pltpu.SemaphoreType.DMA] * NBUF
    + [pltpu.SemaphoreType.DMA] * NBUF,
)
def _emb_gather(idx_hbm, table_hbm, out_hbm, idx_v, rows_v, *sems):
    gsem = sems[:NBUF]
    ssem = sems[NBUF:]
    wid = lax.axis_index("s") * NC + lax.axis_index("c")
    base = wid * PER_W

    pltpu.sync_copy(idx_hbm.at[wid], idx_v)

    for b in range(NBUF):
        pltpu.async_copy(table_hbm.at[idx_v.at[b]], rows_v.at[b], gsem[b])

    @pl.loop(0, NGROUPS)
    def _group(go):
        for b in range(NBUF):
            g = go * NBUF + b
            pltpu.make_async_copy(
                table_hbm.at[idx_v.at[0]], rows_v.at[b], gsem[b]
            ).wait()
            pltpu.async_copy(
                rows_v.at[b],
                out_hbm.at[pl.ds(base + g * CHUNK, CHUNK)],
                ssem[b],
            )

            @pl.when(go < NGROUPS - 1)
            def _():
                pltpu.make_async_copy(
                    rows_v.at[b],
                    out_hbm.at[pl.ds(base, CHUNK)],
                    ssem[b],
                ).wait()
                pltpu.async_copy(
                    table_hbm.at[idx_v.at[g + NBUF]], rows_v.at[b], gsem[b]
                )

    for b in range(NBUF):
        pltpu.make_async_copy(
            rows_v.at[b], out_hbm.at[pl.ds(base, CHUNK)], ssem[b]
        ).wait()


def kernel(idx, emb_weight):
    idx_blocks = idx.reshape(NW, NCH, CHUNK).astype(jnp.int32)
    out = _emb_gather(idx_blocks, emb_weight)
    return out.reshape(B, L, D)
